# baseline (speedup 1.0000x reference)
import functools

import jax
import jax.numpy as jnp
from jax import lax
from jax.experimental import pallas as pl
from jax.experimental.pallas import tpu as pltpu
from jax.experimental.pallas import tpu_sc as plsc

N_NODES = 10000
N_EDGES = 320000
D_FEAT = 128
D_EDGE = 16

NC = 2
NS = 16
NW = NC * NS
NSPLIT = 1
ES = N_EDGES // NSPLIT
EW = ES // NW
C = 64
NCHUNK = N_EDGES // C
KMAX = -(-NCHUNK // NW)
NBUF = 3
RZ = 40
NRC = N_NODES // RZ
NL = 16


def _edge_project(edge_attr, W_edge):
    BE = 16000

    def body(a_ref, w_ref, o_ref):
        o_ref[...] = jnp.dot(a_ref[...], w_ref[...],
                             preferred_element_type=jnp.float32)

    return pl.pallas_call(
        body,
        grid=(ES // BE,),
        in_specs=[
            pl.BlockSpec((BE, D_EDGE), lambda i: (i, 0)),
            pl.BlockSpec((D_EDGE, D_FEAT), lambda i: (0, 0)),
        ],
        out_specs=pl.BlockSpec((BE, D_FEAT), lambda i: (i, 0)),
        out_shape=jax.ShapeDtypeStruct((ES, D_FEAT), jnp.float32),
    )(edge_attr, W_edge)


def _sc_aggregate(x, src, dst, msg):
    mesh = plsc.VectorSubcoreMesh(core_axis_name="c", subcore_axis_name="s")

    @functools.partial(
        pl.kernel,
        out_type=jax.ShapeDtypeStruct((NC, N_NODES, D_FEAT), jnp.float32),
        mesh=mesh,
        scratch_types=[
            [pltpu.VMEM((C,), jnp.int32) for _ in range(NBUF)],
            [pltpu.VMEM((C,), jnp.int32) for _ in range(NBUF)],
            [pltpu.VMEM((C, D_FEAT), jnp.float32) for _ in range(NBUF)],
            pltpu.VMEM_SHARED((N_NODES, D_FEAT), jnp.float32),
            [pltpu.SemaphoreType.DMA for _ in range(NBUF)],
            [pltpu.SemaphoreType.DMA for _ in range(NBUF)],
            [pltpu.SemaphoreType.DMA for _ in range(NBUF)],
        ],
    )
    def agg_kernel(x_hbm, src_hbm, dst_hbm, msg_hbm, out_hbm,
                   sidx, didx, msgb, acc, sem_ld, sem_g, sem_sc):
        cid = lax.axis_index("c")
        sid = lax.axis_index("s")
        wid = sid * NC + cid

        zb = msgb[0].at[pl.ds(0, RZ)]
        def zrow(i, _):
            for j in range(D_FEAT // NL):
                zb[i, pl.ds(j * NL, NL)] = jnp.zeros((NL,), jnp.float32)
            return 0
        lax.fori_loop(0, RZ, zrow, 0)
        for r in range(-(-NRC // NS)):
            cidx = r * NS + sid
            @pl.when(cidx < NRC)
            def _():
                pltpu.sync_copy(zb, acc.at[pl.ds(cidx * RZ, RZ)])
        plsc.subcore_barrier()

        def cix(k):
            return k * NW + wid

        def loads_start(k, b):
            c = cix(k)
            pltpu.async_copy(src_hbm.at[pl.ds(c * C, C)], sidx[b], sem_ld[b])
            pltpu.async_copy(dst_hbm.at[pl.ds(c * C, C)], didx[b], sem_ld[b])
            pltpu.async_copy(msg_hbm.at[pl.ds(c * C, C)], msgb[b], sem_ld[b])

        def loads_wait(k, b):
            c = cix(k)
            pltpu.make_async_copy(src_hbm.at[pl.ds(c * C, C)], sidx[b],
                                  sem_ld[b]).wait()
            pltpu.make_async_copy(dst_hbm.at[pl.ds(c * C, C)], didx[b],
                                  sem_ld[b]).wait()
            pltpu.make_async_copy(msg_hbm.at[pl.ds(c * C, C)], msgb[b],
                                  sem_ld[b]).wait()

        def gather_start(b):
            pltpu.async_copy(x_hbm.at[sidx[b]], msgb[b], sem_g[b], add=True)

        def gather_wait(b):
            pltpu.make_async_copy(x_hbm.at[sidx[b]], msgb[b], sem_g[b]).wait()

        def scatter_start(b):
            pltpu.async_copy(msgb[b], acc.at[didx[b]], sem_sc[b], add=True)

        def scatter_wait(b):
            pltpu.make_async_copy(msgb[b], acc.at[didx[b]], sem_sc[b]).wait()

        loads_start(0, 0)
        loads_wait(0, 0)
        gather_start(0)
        loads_start(1, 1)

        def group(g, _):
            for u in range(NBUF):
                k = g * NBUF + u

                @pl.when(cix(k) < NCHUNK)
                def _():
                    gather_wait(u)

                    def row(i, _):
                        for j in range(D_FEAT // NL):
                            s = pl.ds(j * NL, NL)
                            msgb[u][i, s] = jnp.maximum(msgb[u][i, s], 0.0)
                        return 0
                    lax.fori_loop(0, C, row, 0)
                    scatter_start(u)

                if True:
                    @pl.when(jnp.logical_and(k >= 1, cix(k - 1) < NCHUNK))
                    def _():
                        scatter_wait((u - 1) % NBUF)

                @pl.when(cix(k + 1) < NCHUNK)
                def _():
                    loads_wait(k + 1, (u + 1) % NBUF)
                    gather_start((u + 1) % NBUF)

                @pl.when(cix(k + 2) < NCHUNK)
                def _():
                    loads_start(k + 2, (u + 2) % NBUF)
            return 0
        lax.fori_loop(0, -(-KMAX // NBUF), group, 0)
        plsc.subcore_barrier()

        for r in range(-(-NRC // NS)):
            cidx = r * NS + sid
            @pl.when(cidx < NRC)
            def _():
                rows = pl.ds(cidx * RZ, RZ)
                pltpu.sync_copy(acc.at[rows], zb)
                pltpu.sync_copy(zb, out_hbm.at[cid].at[rows])

    return agg_kernel(x, src, dst, msg)


def _finish(partials, x, W_self, b2d):
    BN = 1000

    def body(*refs):
        p_refs = refs[:NSPLIT]
        x_ref, w_ref, b_ref, o_ref = refs[NSPLIT:]
        acc = p_refs[0][0] + p_refs[0][1]
        for p in p_refs[1:]:
            acc = acc + p[0] + p[1]
        o_ref[...] = (jnp.dot(acc, w_ref[...], preferred_element_type=jnp.float32)
                      + b_ref[...] - x_ref[...])

    return pl.pallas_call(
        body,
        grid=(N_NODES // BN,),
        in_specs=[pl.BlockSpec((NC, BN, D_FEAT), lambda i: (0, i, 0))
                  for _ in range(NSPLIT)] + [
            pl.BlockSpec((BN, D_FEAT), lambda i: (i, 0)),
            pl.BlockSpec((D_FEAT, D_FEAT), lambda i: (0, 0)),
            pl.BlockSpec((1, D_FEAT), lambda i: (0, 0)),
        ],
        out_specs=pl.BlockSpec((BN, D_FEAT), lambda i: (i, 0)),
        out_shape=jax.ShapeDtypeStruct((N_NODES, D_FEAT), jnp.float32),
    )(*partials, x, W_self, b2d)


def kernel(t, x, edge_index, edge_attr, W_edge, W_self, b):
    del t
    src = edge_index[0].astype(jnp.int32)
    dst = edge_index[1].astype(jnp.int32)
    partials = []
    for s in range(NSPLIT):
        sl = slice(s * ES, (s + 1) * ES)
        msg = _edge_project(edge_attr[sl], W_edge)
        partials.append(_sc_aggregate(x, src[sl], dst[sl], msg))
    return _finish(partials, x, W_self, b.reshape(1, D_FEAT))


# speedup vs baseline: 1.2384x; 1.2384x over previous
import functools

import jax
import jax.numpy as jnp
from jax import lax
from jax.experimental import pallas as pl
from jax.experimental.pallas import tpu as pltpu
from jax.experimental.pallas import tpu_sc as plsc

N_NODES = 10000
N_EDGES = 320000
D_FEAT = 128
D_EDGE = 16

NC = 2
NS = 16
NW = NC * NS
NSPLIT = 1
ES = N_EDGES // NSPLIT
EW = ES // NW
C = 40
NCHUNK = EW // C
NBUF = 5
RZ = 40
NRC = N_NODES // RZ
NL = 16


def _edge_project(edge_attr, W_edge):
    BE = 16000

    def body(a_ref, w_ref, o_ref):
        o_ref[...] = jnp.dot(a_ref[...], w_ref[...],
                             preferred_element_type=jnp.float32)

    return pl.pallas_call(
        body,
        grid=(ES // BE,),
        in_specs=[
            pl.BlockSpec((BE, D_EDGE), lambda i: (i, 0)),
            pl.BlockSpec((D_EDGE, D_FEAT), lambda i: (0, 0)),
        ],
        out_specs=pl.BlockSpec((BE, D_FEAT), lambda i: (i, 0)),
        out_shape=jax.ShapeDtypeStruct((ES, D_FEAT), jnp.float32),
    )(edge_attr, W_edge)


def _sc_aggregate(x, src, dst, msg):
    mesh = plsc.VectorSubcoreMesh(core_axis_name="c", subcore_axis_name="s")

    @functools.partial(
        pl.kernel,
        out_type=jax.ShapeDtypeStruct((NC, N_NODES, D_FEAT), jnp.float32),
        mesh=mesh,
        scratch_types=[
            [pltpu.VMEM((C,), jnp.int32) for _ in range(NBUF)],
            [pltpu.VMEM((C,), jnp.int32) for _ in range(NBUF)],
            [pltpu.VMEM((C, D_FEAT), jnp.float32) for _ in range(NBUF)],
            pltpu.VMEM_SHARED((N_NODES, D_FEAT), jnp.float32),
            [pltpu.SemaphoreType.DMA for _ in range(NBUF)],
            [pltpu.SemaphoreType.DMA for _ in range(NBUF)],
            [pltpu.SemaphoreType.DMA for _ in range(NBUF)],
        ],
    )
    def agg_kernel(x_hbm, src_hbm, dst_hbm, msg_hbm, out_hbm,
                   sidx, didx, msgb, acc, sem_ld, sem_g, sem_sc):
        cid = lax.axis_index("c")
        sid = lax.axis_index("s")
        wid = sid * NC + cid
        wbase = wid * EW

        zb = msgb[0]
        def zrow(i, _):
            for j in range(D_FEAT // NL):
                zb[i, pl.ds(j * NL, NL)] = jnp.zeros((NL,), jnp.float32)
            return 0
        lax.fori_loop(0, RZ, zrow, 0)
        for r in range(-(-NRC // NS)):
            cidx = r * NS + sid
            @pl.when(cidx < NRC)
            def _():
                pltpu.sync_copy(zb, acc.at[pl.ds(cidx * RZ, RZ)])
        plsc.subcore_barrier()

        def loads_start(k, b):
            base = wbase + k * C
            pltpu.async_copy(src_hbm.at[pl.ds(base, C)], sidx[b], sem_ld[b])
            pltpu.async_copy(dst_hbm.at[pl.ds(base, C)], didx[b], sem_ld[b])
            pltpu.async_copy(msg_hbm.at[pl.ds(base, C)], msgb[b], sem_ld[b])

        def loads_wait(k, b):
            base = wbase + k * C
            pltpu.make_async_copy(src_hbm.at[pl.ds(base, C)], sidx[b], sem_ld[b]).wait()
            pltpu.make_async_copy(dst_hbm.at[pl.ds(base, C)], didx[b], sem_ld[b]).wait()
            pltpu.make_async_copy(msg_hbm.at[pl.ds(base, C)], msgb[b], sem_ld[b]).wait()

        def gather_start(b):
            pltpu.async_copy(x_hbm.at[sidx[b]], msgb[b], sem_g[b], add=True)

        def gather_wait(b):
            pltpu.make_async_copy(x_hbm.at[sidx[b]], msgb[b], sem_g[b]).wait()

        def scatter_start(b):
            pltpu.async_copy(msgb[b], acc.at[didx[b]], sem_sc[b], add=True)

        def scatter_wait(b):
            pltpu.make_async_copy(msgb[b], acc.at[didx[b]], sem_sc[b]).wait()

        for j in range(NBUF - 1):
            loads_start(j, j)
        for j in range(2):
            loads_wait(j, j)
            gather_start(j)

        def group(g, _):
            for b in range(NBUF):
                k = g * NBUF + b
                bn = (b + NBUF - 1) % NBUF

                @pl.when(k > 0)
                def _():
                    scatter_wait(bn)

                @pl.when(k + NBUF - 1 < NCHUNK)
                def _():
                    loads_start(k + NBUF - 1, bn)

                @pl.when(k + 2 < NCHUNK)
                def _():
                    loads_wait(k + 2, (b + 2) % NBUF)
                    gather_start((b + 2) % NBUF)

                gather_wait(b)

                def row(i, _):
                    for j in range(D_FEAT // NL):
                        s = pl.ds(j * NL, NL)
                        msgb[b][i, s] = jnp.maximum(msgb[b][i, s], 0.0)
                    return 0
                lax.fori_loop(0, C, row, 0)
                scatter_start(b)
            return 0
        lax.fori_loop(0, NCHUNK // NBUF, group, 0)
        scatter_wait((NCHUNK - 1) % NBUF)
        plsc.subcore_barrier()

        for r in range(-(-NRC // NS)):
            cidx = r * NS + sid
            @pl.when(cidx < NRC)
            def _():
                rows = pl.ds(cidx * RZ, RZ)
                pltpu.sync_copy(acc.at[rows], zb)
                pltpu.sync_copy(zb, out_hbm.at[cid].at[rows])

    return agg_kernel(x, src, dst, msg)


def _finish(partials, x, W_self, b2d):
    BN = 1000

    def body(*refs):
        p_refs = refs[:NSPLIT]
        x_ref, w_ref, b_ref, o_ref = refs[NSPLIT:]
        acc = p_refs[0][0] + p_refs[0][1]
        for p in p_refs[1:]:
            acc = acc + p[0] + p[1]
        o_ref[...] = (jnp.dot(acc, w_ref[...], preferred_element_type=jnp.float32)
                      + b_ref[...] - x_ref[...])

    return pl.pallas_call(
        body,
        grid=(N_NODES // BN,),
        in_specs=[pl.BlockSpec((NC, BN, D_FEAT), lambda i: (0, i, 0))
                  for _ in range(NSPLIT)] + [
            pl.BlockSpec((BN, D_FEAT), lambda i: (i, 0)),
            pl.BlockSpec((D_FEAT, D_FEAT), lambda i: (0, 0)),
            pl.BlockSpec((1, D_FEAT), lambda i: (0, 0)),
        ],
        out_specs=pl.BlockSpec((BN, D_FEAT), lambda i: (i, 0)),
        out_shape=jax.ShapeDtypeStruct((N_NODES, D_FEAT), jnp.float32),
    )(*partials, x, W_self, b2d)


def kernel(t, x, edge_index, edge_attr, W_edge, W_self, b):
    del t
    src = edge_index[0].astype(jnp.int32)
    dst = edge_index[1].astype(jnp.int32)
    partials = []
    for s in range(NSPLIT):
        sl = slice(s * ES, (s + 1) * ES)
        msg = _edge_project(edge_attr[sl], W_edge)
        partials.append(_sc_aggregate(x, src[sl], dst[sl], msg))
    return _finish(partials, x, W_self, b.reshape(1, D_FEAT))
